# baseline (speedup 1.0000x reference)
import functools

import jax
import jax.numpy as jnp
from jax import lax
from jax.experimental import pallas as pl
from jax.experimental.pallas import tpu as pltpu
from jax.experimental.pallas import tpu_sc as plsc

USER_NUM = 1000000
ITEM_NUM = 1000000
D = 16
B = 4096
L = 50
LN = 20

NC = 2
NS = 16
NW = NC * NS
CHUNK = 128

POS_ROWS = B * L // NW
NBR_ROWS = B * LN // NW
UID_ROWS = B // NW
POS_CHUNKS = POS_ROWS // CHUNK
NBR_CHUNKS = NBR_ROWS // CHUNK
UID_CHUNKS = UID_ROWS // CHUNK


def _sc_gather_body(uid_hbm, pos_hbm, neg_hbm, nbr_hbm, ut_hbm, it_hbm,
                    user_out, pos_out, neg_out, nbr_out,
                    idx_v, rows_a, rows_b, sem_a, sem_b, sem_w):
    wid = lax.axis_index("s") * NC + lax.axis_index("c")

    def seg(idx3d_hbm, nchunks, table, out_hbm):
        pltpu.sync_copy(idx3d_hbm.at[wid],
                        idx_v.at[pl.ds(0, nchunks)])
        base = wid * nchunks * CHUNK

        def chunk(j, rows, sem):
            cp = pltpu.make_async_copy(table.at[idx_v.at[j]], rows, sem)
            cp.start()
            cp.wait()
            pltpu.sync_copy(rows, out_hbm.at[pl.ds(base + j * CHUNK, CHUNK)])

        def body2(j, _):
            chunk(2 * j, rows_a, sem_a)
            chunk(2 * j + 1, rows_b, sem_b)
            return _
        lax.fori_loop(0, nchunks // 2, body2, 0)
        if nchunks % 2:
            chunk(nchunks - 1, rows_a, sem_a)

    seg(uid_hbm, UID_CHUNKS, ut_hbm, user_out)
    seg(nbr_hbm, NBR_CHUNKS, ut_hbm, nbr_out)
    seg(pos_hbm, POS_CHUNKS, it_hbm, pos_out)
    seg(neg_hbm, POS_CHUNKS, it_hbm, neg_out)
    del sem_w


@jax.jit
def _sc_gather(uid2d, pos2d, neg2d, nbr2d, user_table, item_table):
    mesh = plsc.VectorSubcoreMesh(core_axis_name="c", subcore_axis_name="s")
    f = pl.kernel(
        _sc_gather_body,
        out_type=(
            jax.ShapeDtypeStruct((B, D), jnp.float32),
            jax.ShapeDtypeStruct((B * L, D), jnp.float32),
            jax.ShapeDtypeStruct((B * L, D), jnp.float32),
            jax.ShapeDtypeStruct((B * LN, D), jnp.float32),
        ),
        mesh=mesh,
        compiler_params=pltpu.CompilerParams(use_tc_tiling_on_sc=False),
        scratch_types=[
            pltpu.VMEM((POS_CHUNKS, CHUNK), jnp.int32),
            pltpu.VMEM((CHUNK, D), jnp.float32),
            pltpu.VMEM((CHUNK, D), jnp.float32),
            pltpu.SemaphoreType.DMA,
            pltpu.SemaphoreType.DMA,
            pltpu.SemaphoreType.DMA,
        ],
    )
    return f(uid2d, pos2d, neg2d, nbr2d, user_table, item_table)


def _logits_body(u_ref, p_ref, o_ref):
    u = u_ref[...]
    p = p_ref[...]
    o_ref[...] = jnp.sum(u[:, None, :] * p, axis=-1)


def _logits(user_emb, rows3d, lx, bb=512):
    grid = (B // bb,)
    return pl.pallas_call(
        _logits_body,
        grid=grid,
        in_specs=[
            pl.BlockSpec((bb, D), lambda i: (i, 0)),
            pl.BlockSpec((bb, lx, D), lambda i: (i, 0, 0)),
        ],
        out_specs=pl.BlockSpec((bb, lx), lambda i: (i, 0)),
        out_shape=jax.ShapeDtypeStruct((B, lx), jnp.float32),
    )(user_emb, rows3d)


def kernel(uid, seq, pos, neg, nbr, nbr_iid, user_table, item_table):
    del seq, nbr_iid
    uid2d = uid.astype(jnp.int32).reshape(NW, UID_CHUNKS, CHUNK)
    pos2d = pos.astype(jnp.int32).reshape(NW, POS_CHUNKS, CHUNK)
    neg2d = neg.astype(jnp.int32).reshape(NW, POS_CHUNKS, CHUNK)
    nbr2d = nbr.astype(jnp.int32).reshape(NW, NBR_CHUNKS, CHUNK)

    user_emb, pos_flat, neg_flat, nbr_flat = _sc_gather(
        uid2d, pos2d, neg2d, nbr2d, user_table, item_table)

    pos_hi = pos_flat.reshape(B, L, D)
    neg_hi = neg_flat.reshape(B, L, D)
    nbr_emb = nbr_flat.reshape(B, LN, D)

    pos_logits = _logits(user_emb, pos_hi, L)
    neg_logits = _logits(user_emb, neg_hi, L)
    nbr_logits = _logits(user_emb, nbr_emb, LN)

    return (pos_logits, neg_logits, nbr_logits, user_emb, pos_hi, neg_hi,
            nbr_emb)


# speedup vs baseline: 1.4866x; 1.4866x over previous
import functools

import jax
import jax.numpy as jnp
from jax import lax
from jax.experimental import pallas as pl
from jax.experimental.pallas import tpu as pltpu
from jax.experimental.pallas import tpu_sc as plsc

USER_NUM = 1000000
ITEM_NUM = 1000000
D = 16
B = 4096
L = 50
LN = 20

NC = 2
NS = 16
NW = NC * NS
CHUNK = 128
BPW = B // NW

POS_CHUNKS = BPW * L // CHUNK
NBR_CHUNKS = BPW * LN // CHUNK


def _iota16():
    return lax.iota(jnp.int32, 16)


def _splat(x):
    return jnp.full((16,), x, dtype=jnp.int32)


def _sc_body(uid_hbm, pos_hbm, neg_hbm, nbr_hbm, ut_hbm, it_hbm,
             user_out, pos_out, neg_out, nbr_out, plg_out, nlg_out, blg_out,
             idx_v, rows_v, ptv, ut_v, lbuf, sem_g, sem_w):
    wid = lax.axis_index("s") * NC + lax.axis_index("c")
    i16 = _iota16()

    def gather_chunk(table, j, dst):
        cp = pltpu.make_async_copy(table.at[idx_v.at[j]], dst, sem_g)
        cp.start()
        cp.wait()

    def drain_w(n8, src8, lg_rows, lg_src):
        def d8(_, carry):
            pltpu.make_async_copy(src8.at[pl.ds(0, 8)],
                                  ptv.at[pl.ds(0, 8)], sem_w).wait()
            return carry
        lax.fori_loop(0, n8, d8, 0)
        if lg_rows:
            pltpu.make_async_copy(lg_src.at[pl.ds(0, lg_rows)],
                                  lbuf.at[pl.ds(0, lg_rows)], sem_w).wait()

    pltpu.sync_copy(uid_hbm.at[wid], idx_v.at[pl.ds(0, 1)])
    gather_chunk(ut_hbm, 0, rows_v)

    def urow(r, carry):
        v = plsc.load_gather(rows_v, [_splat(r), i16])
        plsc.store_scatter(ut_v, [i16, _splat(r)], v)
        return carry
    lax.fori_loop(0, CHUNK, urow, 0)

    for ti in range(2):
        cp = pltpu.make_async_copy(
            ut_v.at[pl.ds(ti * 8, 8)],
            user_out.at[pl.ds(ti * 256 + wid * 8, 8)], sem_w)
        cp.start()
    drain_w(2, user_out, 0, None)

    def seg(idx3, nch, ld, table, out_d, out_lg):
        pltpu.sync_copy(idx3.at[wid], idx_v.at[pl.ds(0, nch)])

        def chunk(j, carry):
            gather_chunk(table, j, rows_v)

            def row(r, c2):
                fr = j * CHUNK + r
                b_l = fr // ld
                ll = fr % ld
                v = plsc.load_gather(rows_v, [_splat(r), i16])
                plsc.store_scatter(ptv, [ll * 16 + i16, _splat(b_l)], v)
                return c2
            lax.fori_loop(0, CHUNK, row, 0)
            return carry
        lax.fori_loop(0, nch, chunk, 0)

        def lg_l(l, carry):
            def lg_g(g, c2):
                g16 = g * 16 + i16
                acc = jnp.zeros((16,), jnp.float32)
                for d in range(D):
                    pv = plsc.load_gather(ptv, [_splat(l * 16 + d), g16])
                    uv = plsc.load_gather(ut_v, [_splat(d), g16])
                    acc = acc + pv * uv
                plsc.store_scatter(lbuf, [_splat(l), g16], acc)
                return c2
            lax.fori_loop(0, 8, lg_g, 0)
            return carry
        lax.fori_loop(0, ld, lg_l, 0)

        def wr_l(l, carry):
            for ti in range(2):
                cp = pltpu.make_async_copy(
                    ptv.at[pl.ds(l * 16 + ti * 8, 8)],
                    out_d.at[pl.ds((l * 2 + ti) * 256 + wid * 8, 8)], sem_w)
                cp.start()
            return carry
        lax.fori_loop(0, ld, wr_l, 0)
        cp = pltpu.make_async_copy(lbuf.at[pl.ds(0, ld)],
                                   out_lg.at[pl.ds(wid * ld, ld)], sem_w)
        cp.start()
        drain_w(2 * ld, out_d, ld, out_lg)

    seg(nbr_hbm, NBR_CHUNKS, LN, ut_hbm, nbr_out, blg_out)
    seg(pos_hbm, POS_CHUNKS, L, it_hbm, pos_out, plg_out)
    seg(neg_hbm, POS_CHUNKS, L, it_hbm, neg_out, nlg_out)


@jax.jit
def _sc_fused(uid3, pos3, neg3, nbr3, user_table, item_table):
    mesh = plsc.VectorSubcoreMesh(core_axis_name="c", subcore_axis_name="s")
    f = pl.kernel(
        _sc_body,
        out_type=(
            jax.ShapeDtypeStruct((2 * 32 * 8, 128), jnp.float32),
            jax.ShapeDtypeStruct((L * 2 * 32 * 8, 128), jnp.float32),
            jax.ShapeDtypeStruct((L * 2 * 32 * 8, 128), jnp.float32),
            jax.ShapeDtypeStruct((LN * 2 * 32 * 8, 128), jnp.float32),
            jax.ShapeDtypeStruct((NW * L, 128), jnp.float32),
            jax.ShapeDtypeStruct((NW * L, 128), jnp.float32),
            jax.ShapeDtypeStruct((NW * LN, 128), jnp.float32),
        ),
        mesh=mesh,
        compiler_params=pltpu.CompilerParams(use_tc_tiling_on_sc=False,
                                             needs_layout_passes=False),
        scratch_types=[
            pltpu.VMEM((POS_CHUNKS, CHUNK), jnp.int32),
            pltpu.VMEM((CHUNK, D), jnp.float32),
            pltpu.VMEM((L * 16, 128), jnp.float32),
            pltpu.VMEM((16, 128), jnp.float32),
            pltpu.VMEM((L, 128), jnp.float32),
            pltpu.SemaphoreType.DMA,
            pltpu.SemaphoreType.DMA,
        ],
    )
    return f(uid3, pos3, neg3, nbr3, user_table, item_table)


def kernel(uid, seq, pos, neg, nbr, nbr_iid, user_table, item_table):
    del seq, nbr_iid
    uid3 = uid.astype(jnp.int32).reshape(NW, 1, CHUNK)
    pos3 = pos.astype(jnp.int32).reshape(NW, POS_CHUNKS, CHUNK)
    neg3 = neg.astype(jnp.int32).reshape(NW, POS_CHUNKS, CHUNK)
    nbr3 = nbr.astype(jnp.int32).reshape(NW, NBR_CHUNKS, CHUNK)

    user_o, pos_o, neg_o, nbr_o, plg, nlg, blg = _sc_fused(
        uid3, pos3, neg3, nbr3, user_table, item_table)

    user_emb = user_o.reshape(2, 32, 8, 128).transpose(1, 3, 0, 2) \
        .reshape(B, D)
    pos_hi = pos_o.reshape(L, 2, 32, 8, 128).transpose(2, 4, 0, 1, 3) \
        .reshape(B, L, D)
    neg_hi = neg_o.reshape(L, 2, 32, 8, 128).transpose(2, 4, 0, 1, 3) \
        .reshape(B, L, D)
    nbr_emb = nbr_o.reshape(LN, 2, 32, 8, 128).transpose(2, 4, 0, 1, 3) \
        .reshape(B, LN, D)
    pos_logits = plg.reshape(NW, L, 128).transpose(0, 2, 1).reshape(B, L)
    neg_logits = nlg.reshape(NW, L, 128).transpose(0, 2, 1).reshape(B, L)
    nbr_logits = blg.reshape(NW, LN, 128).transpose(0, 2, 1).reshape(B, LN)

    return (pos_logits, neg_logits, nbr_logits, user_emb, pos_hi, neg_hi,
            nbr_emb)


# speedup vs baseline: 1.5009x; 1.0096x over previous
import functools

import jax
import jax.numpy as jnp
from jax import lax
from jax.experimental import pallas as pl
from jax.experimental.pallas import tpu as pltpu
from jax.experimental.pallas import tpu_sc as plsc

USER_NUM = 1000000
ITEM_NUM = 1000000
D = 16
B = 4096
L = 50
LN = 20

NC = 2
NS = 16
NW = NC * NS
CHUNK = 128
BPW = B // NW

POS_CHUNKS = BPW * L // CHUNK
NBR_CHUNKS = BPW * LN // CHUNK


def _iota16():
    return lax.iota(jnp.int32, 16)


def _splat(x):
    return jnp.full((16,), x, dtype=jnp.int32)


NROW = USER_NUM
NSLAB = NROW // CHUNK
TAIL = NROW - NSLAB * CHUNK
SPW = NSLAB // NW
SREM = NSLAB - SPW * NW


def _tr_body(ut_t, it_t, u2, i2, slab0, slab1, ob0, ob1, tbuf,
             gs0, gs1, ws0, ws1, ts):
    wid = lax.axis_index("s") * NC + lax.axis_index("c")
    i16 = _iota16()
    slabs = [slab0, slab1]
    obs = [ob0, ob1]
    gsems = [gs0, gs1]
    wsems = [ws0, ws1]

    def transpose_slab(slab, ob):
        def row(r, c):
            v = plsc.load_gather(slab, [i16, _splat(r)])
            plsc.store_scatter(ob, [_splat(r >> 3), (r & 7) * 16 + i16], v)
            return c
        lax.fori_loop(0, CHUNK, row, 0)

    def one_table(src, dst):
        base = wid * SPW

        def g_start(k, b):
            pltpu.make_async_copy(
                src.at[:, pl.ds((base + k) * CHUNK, CHUNK)],
                slabs[b], gsems[b]).start()

        def w_start(k, b):
            pltpu.make_async_copy(
                obs[b], dst.at[pl.ds((base + k) * 16, 16)], wsems[b]).start()

        g_start(0, 0)
        g_start(1, 1)

        def pair(t, carry):
            for b in range(2):
                k = 2 * t + b

                @pl.when(t >= 1)
                def _():
                    pltpu.make_async_copy(obs[b], dst.at[pl.ds(0, 16)],
                                          wsems[b]).wait()
                pltpu.make_async_copy(src.at[:, pl.ds(0, CHUNK)],
                                      slabs[b], gsems[b]).wait()
                transpose_slab(slabs[b], obs[b])
                w_start(k, b)

                @pl.when(k + 2 < SPW)
                def _():
                    g_start(k + 2, b)
            return carry
        lax.fori_loop(0, SPW // 2, pair, 0)
        for b in range(2):
            pltpu.make_async_copy(obs[b], dst.at[pl.ds(0, 16)],
                                  wsems[b]).wait()

        @pl.when(wid < SREM)
        def _():
            k = NW * SPW + wid
            cp = pltpu.make_async_copy(
                src.at[:, pl.ds(k * CHUNK, CHUNK)], slabs[0], gsems[0])
            cp.start()
            cp.wait()
            transpose_slab(slabs[0], obs[0])
            cp2 = pltpu.make_async_copy(
                obs[0], dst.at[pl.ds(k * 16, 16)], wsems[0])
            cp2.start()
            cp2.wait()

        @pl.when(wid == 0)
        def _():
            cp = pltpu.make_async_copy(
                src.at[:, pl.ds(NSLAB * CHUNK, TAIL)], tbuf, ts)
            cp.start()
            cp.wait()

            def trow(r, c):
                v = plsc.load_gather(tbuf, [i16, _splat(r)])
                plsc.store_scatter(obs[0],
                                   [_splat(r >> 3), (r & 7) * 16 + i16], v)
                return c
            lax.fori_loop(0, TAIL, trow, 0)
            cp2 = pltpu.make_async_copy(
                obs[0].at[pl.ds(0, 8)],
                dst.at[pl.ds(NSLAB * 16, 8)], wsems[0])
            cp2.start()
            cp2.wait()

    one_table(ut_t, u2)
    one_table(it_t, i2)


@jax.jit
def _sc_transpose(ut_t, it_t):
    mesh = plsc.VectorSubcoreMesh(core_axis_name="c", subcore_axis_name="s")
    f = pl.kernel(
        _tr_body,
        out_type=(
            jax.ShapeDtypeStruct((NROW // 8, 128), jnp.float32),
            jax.ShapeDtypeStruct((NROW // 8, 128), jnp.float32),
        ),
        mesh=mesh,
        compiler_params=pltpu.CompilerParams(use_tc_tiling_on_sc=True,
                                             needs_layout_passes=False),
        scratch_types=[
            pltpu.VMEM((16, CHUNK), jnp.float32),
            pltpu.VMEM((16, CHUNK), jnp.float32),
            pltpu.VMEM((16, CHUNK), jnp.float32),
            pltpu.VMEM((16, CHUNK), jnp.float32),
            pltpu.VMEM((16, TAIL), jnp.float32),
            pltpu.SemaphoreType.DMA,
            pltpu.SemaphoreType.DMA,
            pltpu.SemaphoreType.DMA,
            pltpu.SemaphoreType.DMA,
            pltpu.SemaphoreType.DMA,
        ],
    )
    return f(ut_t, it_t)


def _sc_body(uid_hbm, pos_hbm, neg_hbm, nbr_hbm, ut_hbm, it_hbm,
             user_out, pos_out, neg_out, nbr_out, plg_out, nlg_out, blg_out,
             idx_v, rows_v, ptv, ut_v, lbuf, sem_g, sem_w):
    wid = lax.axis_index("s") * NC + lax.axis_index("c")
    i16 = _iota16()

    def gather_chunk(table, j, dst):
        cp = pltpu.make_async_copy(table.at[idx_v.at[j]], dst, sem_g)
        cp.start()
        cp.wait()

    def drain_w(n8, src8, lg_rows, lg_src):
        def d8(_, carry):
            pltpu.make_async_copy(src8.at[pl.ds(0, 8)],
                                  ptv.at[pl.ds(0, 8)], sem_w).wait()
            return carry
        lax.fori_loop(0, n8, d8, 0)
        if lg_rows:
            pltpu.make_async_copy(lg_src.at[pl.ds(0, lg_rows)],
                                  lbuf.at[pl.ds(0, lg_rows)], sem_w).wait()

    pltpu.sync_copy(uid_hbm.at[wid], idx_v.at[pl.ds(0, 1)])
    gather_chunk(ut_hbm, 0, rows_v)

    def urow(r, carry):
        v = plsc.load_gather(rows_v, [_splat(r), i16])
        plsc.store_scatter(ut_v, [i16, _splat(r)], v)
        return carry
    lax.fori_loop(0, CHUNK, urow, 0)

    for ti in range(2):
        cp = pltpu.make_async_copy(
            ut_v.at[pl.ds(ti * 8, 8)],
            user_out.at[pl.ds(ti * 256 + wid * 8, 8)], sem_w)
        cp.start()
    drain_w(2, user_out, 0, None)

    def seg(idx3, nch, ld, table, out_d, out_lg):
        pltpu.sync_copy(idx3.at[wid], idx_v.at[pl.ds(0, nch)])

        def chunk(j, carry):
            gather_chunk(table, j, rows_v)

            def row(r, c2):
                fr = j * CHUNK + r
                b_l = fr // ld
                ll = fr % ld
                v = plsc.load_gather(rows_v, [_splat(r), i16])
                plsc.store_scatter(ptv, [ll * 16 + i16, _splat(b_l)], v)
                return c2
            lax.fori_loop(0, CHUNK, row, 0)
            return carry
        lax.fori_loop(0, nch, chunk, 0)

        def lg_l(l, carry):
            def lg_g(g, c2):
                g16 = g * 16 + i16
                acc = jnp.zeros((16,), jnp.float32)
                for d in range(D):
                    pv = plsc.load_gather(ptv, [_splat(l * 16 + d), g16])
                    uv = plsc.load_gather(ut_v, [_splat(d), g16])
                    acc = acc + pv * uv
                plsc.store_scatter(lbuf, [_splat(l), g16], acc)
                return c2
            lax.fori_loop(0, 8, lg_g, 0)
            return carry
        lax.fori_loop(0, ld, lg_l, 0)

        def wr_l(l, carry):
            for ti in range(2):
                cp = pltpu.make_async_copy(
                    ptv.at[pl.ds(l * 16 + ti * 8, 8)],
                    out_d.at[pl.ds((l * 2 + ti) * 256 + wid * 8, 8)], sem_w)
                cp.start()
            return carry
        lax.fori_loop(0, ld, wr_l, 0)
        cp = pltpu.make_async_copy(lbuf.at[pl.ds(0, ld)],
                                   out_lg.at[pl.ds(wid * ld, ld)], sem_w)
        cp.start()
        drain_w(2 * ld, out_d, ld, out_lg)

    seg(nbr_hbm, NBR_CHUNKS, LN, ut_hbm, nbr_out, blg_out)
    seg(pos_hbm, POS_CHUNKS, L, it_hbm, pos_out, plg_out)
    seg(neg_hbm, POS_CHUNKS, L, it_hbm, neg_out, nlg_out)


@jax.jit
def _sc_fused(uid3, pos3, neg3, nbr3, user_table, item_table):
    mesh = plsc.VectorSubcoreMesh(core_axis_name="c", subcore_axis_name="s")
    f = pl.kernel(
        _sc_body,
        out_type=(
            jax.ShapeDtypeStruct((2 * 32 * 8, 128), jnp.float32),
            jax.ShapeDtypeStruct((L * 2 * 32 * 8, 128), jnp.float32),
            jax.ShapeDtypeStruct((L * 2 * 32 * 8, 128), jnp.float32),
            jax.ShapeDtypeStruct((LN * 2 * 32 * 8, 128), jnp.float32),
            jax.ShapeDtypeStruct((NW * L, 128), jnp.float32),
            jax.ShapeDtypeStruct((NW * L, 128), jnp.float32),
            jax.ShapeDtypeStruct((NW * LN, 128), jnp.float32),
        ),
        mesh=mesh,
        compiler_params=pltpu.CompilerParams(use_tc_tiling_on_sc=False,
                                             needs_layout_passes=False),
        scratch_types=[
            pltpu.VMEM((POS_CHUNKS, CHUNK), jnp.int32),
            pltpu.VMEM((CHUNK, D), jnp.float32),
            pltpu.VMEM((L * 16, 128), jnp.float32),
            pltpu.VMEM((16, 128), jnp.float32),
            pltpu.VMEM((L, 128), jnp.float32),
            pltpu.SemaphoreType.DMA,
            pltpu.SemaphoreType.DMA,
        ],
    )
    return f(uid3, pos3, neg3, nbr3, user_table, item_table)


def kernel(uid, seq, pos, neg, nbr, nbr_iid, user_table, item_table):
    del seq, nbr_iid
    uid3 = uid.astype(jnp.int32).reshape(NW, 1, CHUNK)
    pos3 = pos.astype(jnp.int32).reshape(NW, POS_CHUNKS, CHUNK)
    neg3 = neg.astype(jnp.int32).reshape(NW, POS_CHUNKS, CHUNK)
    nbr3 = nbr.astype(jnp.int32).reshape(NW, NBR_CHUNKS, CHUNK)

    u2, i2 = _sc_transpose(user_table.T, item_table.T)
    ut_lin = u2.reshape(USER_NUM, D)
    it_lin = i2.reshape(ITEM_NUM, D)

    user_o, pos_o, neg_o, nbr_o, plg, nlg, blg = _sc_fused(
        uid3, pos3, neg3, nbr3, ut_lin, it_lin)

    user_emb = user_o.reshape(2, 32, 8, 128).transpose(1, 3, 0, 2) \
        .reshape(B, D)
    pos_hi = pos_o.reshape(L, 2, 32, 8, 128).transpose(2, 4, 0, 1, 3) \
        .reshape(B, L, D)
    neg_hi = neg_o.reshape(L, 2, 32, 8, 128).transpose(2, 4, 0, 1, 3) \
        .reshape(B, L, D)
    nbr_emb = nbr_o.reshape(LN, 2, 32, 8, 128).transpose(2, 4, 0, 1, 3) \
        .reshape(B, LN, D)
    pos_logits = plg.reshape(NW, L, 128).transpose(0, 2, 1).reshape(B, L)
    neg_logits = nlg.reshape(NW, L, 128).transpose(0, 2, 1).reshape(B, L)
    nbr_logits = blg.reshape(NW, LN, 128).transpose(0, 2, 1).reshape(B, LN)

    return (pos_logits, neg_logits, nbr_logits, user_emb, pos_hi, neg_hi,
            nbr_emb)


# speedup vs baseline: 2.7737x; 1.8480x over previous
import functools

import jax
import jax.numpy as jnp
from jax import lax
from jax.experimental import pallas as pl
from jax.experimental.pallas import tpu as pltpu
from jax.experimental.pallas import tpu_sc as plsc

USER_NUM = 1000000
ITEM_NUM = 1000000
D = 16
B = 4096
L = 50
LN = 20

NC = 2
NS = 16
NW = NC * NS
CHUNK = 128
BPW = B // NW

POS_CHUNKS = BPW * L // CHUNK
NBR_CHUNKS = BPW * LN // CHUNK


def _iota16():
    return lax.iota(jnp.int32, 16)


def _splat(x):
    return jnp.full((16,), x, dtype=jnp.int32)


NROW = USER_NUM
NSLAB = NROW // CHUNK
TAIL = NROW - NSLAB * CHUNK
SPW = NSLAB // NW
SREM = NSLAB - SPW * NW


def _tr_body(ut_t, it_t, u2, i2, slab0, slab1, ob0, ob1, tbuf,
             gs0, gs1, ws0, ws1, ts):
    wid = lax.axis_index("s") * NC + lax.axis_index("c")
    i16 = _iota16()
    slabs = [slab0, slab1]
    obs = [ob0, ob1]
    gsems = [gs0, gs1]
    wsems = [ws0, ws1]

    rs = [(i16 * 16 + d) >> 7 for d in range(D)]
    cs = [(i16 * 16 + d) & 127 for d in range(D)]

    def transpose_slab(slab, ob):
        def grp(g, c):
            r2 = g * 2
            for d in range(D):
                v = slab.at[d][pl.ds(g * 16, 16)]
                plsc.store_scatter(ob, [r2 + rs[d], cs[d]], v)
            return c
        lax.fori_loop(0, 8, grp, 0)

    def one_table(src, dst):
        base = wid * SPW

        def g_start(k, b):
            pltpu.make_async_copy(
                src.at[:, pl.ds((base + k) * CHUNK, CHUNK)],
                slabs[b], gsems[b]).start()

        def w_start(k, b):
            pltpu.make_async_copy(
                obs[b], dst.at[pl.ds((base + k) * 16, 16)], wsems[b]).start()

        g_start(0, 0)
        g_start(1, 1)

        def pair(t, carry):
            for b in range(2):
                k = 2 * t + b

                @pl.when(t >= 1)
                def _():
                    pltpu.make_async_copy(obs[b], dst.at[pl.ds(0, 16)],
                                          wsems[b]).wait()
                pltpu.make_async_copy(src.at[:, pl.ds(0, CHUNK)],
                                      slabs[b], gsems[b]).wait()
                transpose_slab(slabs[b], obs[b])
                w_start(k, b)

                @pl.when(k + 2 < SPW)
                def _():
                    g_start(k + 2, b)
            return carry
        lax.fori_loop(0, SPW // 2, pair, 0)
        for b in range(2):
            pltpu.make_async_copy(obs[b], dst.at[pl.ds(0, 16)],
                                  wsems[b]).wait()

        @pl.when(wid < SREM)
        def _():
            k = NW * SPW + wid
            cp = pltpu.make_async_copy(
                src.at[:, pl.ds(k * CHUNK, CHUNK)], slabs[0], gsems[0])
            cp.start()
            cp.wait()
            transpose_slab(slabs[0], obs[0])
            cp2 = pltpu.make_async_copy(
                obs[0], dst.at[pl.ds(k * 16, 16)], wsems[0])
            cp2.start()
            cp2.wait()

        @pl.when(wid == 0)
        def _():
            cp = pltpu.make_async_copy(
                src.at[:, pl.ds(NSLAB * CHUNK, TAIL)], tbuf, ts)
            cp.start()
            cp.wait()

            def trow(r, c):
                v = plsc.load_gather(tbuf, [i16, _splat(r)])
                plsc.store_scatter(obs[0],
                                   [_splat(r >> 3), (r & 7) * 16 + i16], v)
                return c
            lax.fori_loop(0, TAIL, trow, 0)
            cp2 = pltpu.make_async_copy(
                obs[0].at[pl.ds(0, 8)],
                dst.at[pl.ds(NSLAB * 16, 8)], wsems[0])
            cp2.start()
            cp2.wait()

    one_table(ut_t, u2)
    one_table(it_t, i2)


@jax.jit
def _sc_transpose(ut_t, it_t):
    mesh = plsc.VectorSubcoreMesh(core_axis_name="c", subcore_axis_name="s")
    f = pl.kernel(
        _tr_body,
        out_type=(
            jax.ShapeDtypeStruct((NROW // 8, 128), jnp.float32),
            jax.ShapeDtypeStruct((NROW // 8, 128), jnp.float32),
        ),
        mesh=mesh,
        compiler_params=pltpu.CompilerParams(use_tc_tiling_on_sc=True,
                                             needs_layout_passes=False),
        scratch_types=[
            pltpu.VMEM((16, CHUNK), jnp.float32),
            pltpu.VMEM((16, CHUNK), jnp.float32),
            pltpu.VMEM((16, CHUNK), jnp.float32),
            pltpu.VMEM((16, CHUNK), jnp.float32),
            pltpu.VMEM((16, TAIL), jnp.float32),
            pltpu.SemaphoreType.DMA,
            pltpu.SemaphoreType.DMA,
            pltpu.SemaphoreType.DMA,
            pltpu.SemaphoreType.DMA,
            pltpu.SemaphoreType.DMA,
        ],
    )
    return f(ut_t, it_t)


def _sc_body(uid_hbm, pos_hbm, neg_hbm, nbr_hbm, ut_hbm, it_hbm,
             user_out, pos_out, neg_out, nbr_out, plg_out, nlg_out, blg_out,
             idx_v, rows_a, rows_b, ptv, ut_v, lbuf, sem_ga, sem_gb, sem_w):
    wid = lax.axis_index("s") * NC + lax.axis_index("c")
    i16 = _iota16()
    bufs = [rows_a, rows_b]
    gsems = [sem_ga, sem_gb]

    def g_start(table, j, b):
        pltpu.make_async_copy(table.at[idx_v.at[j]], bufs[b],
                              gsems[b]).start()

    def g_wait(table, b):
        pltpu.make_async_copy(table.at[idx_v.at[0]], bufs[b],
                              gsems[b]).wait()

    def drain_w(n8, src8, lg_rows, lg_src):
        def d8(_, carry):
            pltpu.make_async_copy(src8.at[pl.ds(0, 8)],
                                  ptv.at[pl.ds(0, 8)], sem_w).wait()
            return carry
        lax.fori_loop(0, n8, d8, 0)
        if lg_rows:
            pltpu.make_async_copy(lg_src.at[pl.ds(0, lg_rows)],
                                  lbuf.at[pl.ds(0, lg_rows)], sem_w).wait()

    pltpu.sync_copy(uid_hbm.at[wid], idx_v.at[pl.ds(0, 1)])
    g_start(ut_hbm, 0, 0)
    g_wait(ut_hbm, 0)

    def ugrp(g, carry):
        col = g * 16 + i16
        for d in range(D):
            v = plsc.load_gather(rows_a, [col, _splat(d)])
            plsc.store_scatter(ut_v, [_splat(d), col], v)
        return carry
    lax.fori_loop(0, 8, ugrp, 0)

    for ti in range(2):
        cp = pltpu.make_async_copy(
            ut_v.at[pl.ds(ti * 8, 8)],
            user_out.at[pl.ds(ti * 256 + wid * 8, 8)], sem_w)
        cp.start()
    drain_w(2, user_out, 0, None)

    def seg(idx3, nch, ld, table, out_d, out_lg):
        pltpu.sync_copy(idx3.at[wid], idx_v.at[pl.ds(0, nch)])
        g_start(table, 0, 0)

        def process(j, rv):
            def grp(g, c2):
                col = g * 16 + i16
                fr16 = j * CHUNK + col
                bl16 = fr16 // ld
                ll16 = (fr16 % ld) * 16
                for d in range(D):
                    v = plsc.load_gather(rv, [col, _splat(d)])
                    plsc.store_scatter(ptv, [ll16 + d, bl16], v)
                return c2
            lax.fori_loop(0, 8, grp, 0)

        def pair(t, carry):
            for b in range(2):
                j = 2 * t + b

                @pl.when(j + 1 < nch)
                def _():
                    g_start(table, j + 1, 1 - b)
                g_wait(table, b)
                process(j, bufs[b])
            return carry
        lax.fori_loop(0, nch // 2, pair, 0)

        def lg_l(l, carry):
            def lg_g(g, c2):
                g16 = g * 16 + i16
                acc = jnp.zeros((16,), jnp.float32)
                for d in range(D):
                    pv = plsc.load_gather(ptv, [_splat(l * 16 + d), g16])
                    uv = plsc.load_gather(ut_v, [_splat(d), g16])
                    acc = acc + pv * uv
                plsc.store_scatter(lbuf, [_splat(l), g16], acc)
                return c2
            lax.fori_loop(0, 8, lg_g, 0)
            return carry
        lax.fori_loop(0, ld, lg_l, 0)

        def wr_l(l, carry):
            for ti in range(2):
                cp = pltpu.make_async_copy(
                    ptv.at[pl.ds(l * 16 + ti * 8, 8)],
                    out_d.at[pl.ds((l * 2 + ti) * 256 + wid * 8, 8)], sem_w)
                cp.start()
            return carry
        lax.fori_loop(0, ld, wr_l, 0)
        cp = pltpu.make_async_copy(lbuf.at[pl.ds(0, ld)],
                                   out_lg.at[pl.ds(wid * ld, ld)], sem_w)
        cp.start()
        drain_w(2 * ld, out_d, ld, out_lg)

    seg(nbr_hbm, NBR_CHUNKS, LN, ut_hbm, nbr_out, blg_out)
    seg(pos_hbm, POS_CHUNKS, L, it_hbm, pos_out, plg_out)
    seg(neg_hbm, POS_CHUNKS, L, it_hbm, neg_out, nlg_out)


@jax.jit
def _sc_fused(uid3, pos3, neg3, nbr3, user_table, item_table):
    mesh = plsc.VectorSubcoreMesh(core_axis_name="c", subcore_axis_name="s")
    f = pl.kernel(
        _sc_body,
        out_type=(
            jax.ShapeDtypeStruct((2 * 32 * 8, 128), jnp.float32),
            jax.ShapeDtypeStruct((L * 2 * 32 * 8, 128), jnp.float32),
            jax.ShapeDtypeStruct((L * 2 * 32 * 8, 128), jnp.float32),
            jax.ShapeDtypeStruct((LN * 2 * 32 * 8, 128), jnp.float32),
            jax.ShapeDtypeStruct((NW * L, 128), jnp.float32),
            jax.ShapeDtypeStruct((NW * L, 128), jnp.float32),
            jax.ShapeDtypeStruct((NW * LN, 128), jnp.float32),
        ),
        mesh=mesh,
        compiler_params=pltpu.CompilerParams(use_tc_tiling_on_sc=False,
                                             needs_layout_passes=False),
        scratch_types=[
            pltpu.VMEM((POS_CHUNKS, CHUNK), jnp.int32),
            pltpu.VMEM((CHUNK, D), jnp.float32),
            pltpu.VMEM((CHUNK, D), jnp.float32),
            pltpu.VMEM((L * 16, 128), jnp.float32),
            pltpu.VMEM((16, 128), jnp.float32),
            pltpu.VMEM((L, 128), jnp.float32),
            pltpu.SemaphoreType.DMA,
            pltpu.SemaphoreType.DMA,
            pltpu.SemaphoreType.DMA,
        ],
    )
    return f(uid3, pos3, neg3, nbr3, user_table, item_table)


def kernel(uid, seq, pos, neg, nbr, nbr_iid, user_table, item_table):
    del seq, nbr_iid
    uid3 = uid.astype(jnp.int32).reshape(NW, 1, CHUNK)
    pos3 = pos.astype(jnp.int32).reshape(NW, POS_CHUNKS, CHUNK)
    neg3 = neg.astype(jnp.int32).reshape(NW, POS_CHUNKS, CHUNK)
    nbr3 = nbr.astype(jnp.int32).reshape(NW, NBR_CHUNKS, CHUNK)

    u2, i2 = _sc_transpose(user_table.T, item_table.T)
    ut_lin = u2.reshape(USER_NUM, D)
    it_lin = i2.reshape(ITEM_NUM, D)

    user_o, pos_o, neg_o, nbr_o, plg, nlg, blg = _sc_fused(
        uid3, pos3, neg3, nbr3, ut_lin, it_lin)

    user_emb = user_o.reshape(2, 32, 8, 128).transpose(1, 3, 0, 2) \
        .reshape(B, D)
    pos_hi = pos_o.reshape(L, 2, 32, 8, 128).transpose(2, 4, 0, 1, 3) \
        .reshape(B, L, D)
    neg_hi = neg_o.reshape(L, 2, 32, 8, 128).transpose(2, 4, 0, 1, 3) \
        .reshape(B, L, D)
    nbr_emb = nbr_o.reshape(LN, 2, 32, 8, 128).transpose(2, 4, 0, 1, 3) \
        .reshape(B, LN, D)
    pos_logits = plg.reshape(NW, L, 128).transpose(0, 2, 1).reshape(B, L)
    neg_logits = nlg.reshape(NW, L, 128).transpose(0, 2, 1).reshape(B, L)
    nbr_logits = blg.reshape(NW, LN, 128).transpose(0, 2, 1).reshape(B, LN)

    return (pos_logits, neg_logits, nbr_logits, user_emb, pos_hi, neg_hi,
            nbr_emb)


# speedup vs baseline: 3.2932x; 1.1873x over previous
import functools

import jax
import jax.numpy as jnp
from jax import lax
from jax.experimental import pallas as pl
from jax.experimental.pallas import tpu as pltpu
from jax.experimental.pallas import tpu_sc as plsc

USER_NUM = 1000000
ITEM_NUM = 1000000
D = 16
B = 4096
L = 50
LN = 20

NC = 2
NS = 16
NW = NC * NS
CHUNK = 128
BPW = B // NW

POS_CHUNKS = BPW * L // CHUNK
NBR_CHUNKS = BPW * LN // CHUNK


def _iota16():
    return lax.iota(jnp.int32, 16)


def _splat(x):
    return jnp.full((16,), x, dtype=jnp.int32)


NROW = USER_NUM
NSLAB = NROW // CHUNK
TAIL = NROW - NSLAB * CHUNK
SPW = NSLAB // NW
SREM = NSLAB - SPW * NW


NBUF = 4


def _tr_body(ut_t, it_t, u2, i2, slab0, slab1, slab2, slab3,
             ob0, ob1, ob2, ob3, tbuf,
             gs0, gs1, gs2, gs3, ws0, ws1, ws2, ws3, ts):
    wid = lax.axis_index("s") * NC + lax.axis_index("c")
    i16 = _iota16()
    slabs = [slab0, slab1, slab2, slab3]
    obs = [ob0, ob1, ob2, ob3]
    gsems = [gs0, gs1, gs2, gs3]
    wsems = [ws0, ws1, ws2, ws3]

    rs = [(i16 * 16 + d) >> 7 for d in range(D)]
    cs = [(i16 * 16 + d) & 127 for d in range(D)]

    def transpose_slab(slab, ob):
        def grp(g, c):
            r2 = g * 2
            for d in range(D):
                v = slab.at[d][pl.ds(g * 16, 16)]
                plsc.store_scatter(ob, [r2 + rs[d], cs[d]], v)
            return c
        lax.fori_loop(0, 8, grp, 0)

    def one_table(src, dst):
        base = wid * SPW

        def g_start(k, b):
            pltpu.make_async_copy(
                src.at[:, pl.ds((base + k) * CHUNK, CHUNK)],
                slabs[b], gsems[b]).start()

        def w_start(k, b):
            pltpu.make_async_copy(
                obs[b], dst.at[pl.ds((base + k) * 16, 16)], wsems[b]).start()

        for b in range(NBUF):
            g_start(b, b)

        def quad(t, carry):
            for b in range(NBUF):
                k = NBUF * t + b

                @pl.when(t >= 1)
                def _():
                    pltpu.make_async_copy(obs[b], dst.at[pl.ds(0, 16)],
                                          wsems[b]).wait()
                pltpu.make_async_copy(src.at[:, pl.ds(0, CHUNK)],
                                      slabs[b], gsems[b]).wait()
                transpose_slab(slabs[b], obs[b])
                w_start(k, b)

                @pl.when(k + NBUF < SPW)
                def _():
                    g_start(k + NBUF, b)
            return carry
        lax.fori_loop(0, SPW // NBUF, quad, 0)
        for b in range(NBUF):
            pltpu.make_async_copy(obs[b], dst.at[pl.ds(0, 16)],
                                  wsems[b]).wait()

        @pl.when(wid < SREM)
        def _():
            k = NW * SPW + wid
            cp = pltpu.make_async_copy(
                src.at[:, pl.ds(k * CHUNK, CHUNK)], slabs[0], gsems[0])
            cp.start()
            cp.wait()
            transpose_slab(slabs[0], obs[0])
            cp2 = pltpu.make_async_copy(
                obs[0], dst.at[pl.ds(k * 16, 16)], wsems[0])
            cp2.start()
            cp2.wait()

        @pl.when(wid == 0)
        def _():
            cp = pltpu.make_async_copy(
                src.at[:, pl.ds(NSLAB * CHUNK, TAIL)], tbuf, ts)
            cp.start()
            cp.wait()

            def trow(r, c):
                v = plsc.load_gather(tbuf, [i16, _splat(r)])
                plsc.store_scatter(obs[0],
                                   [_splat(r >> 3), (r & 7) * 16 + i16], v)
                return c
            lax.fori_loop(0, TAIL, trow, 0)
            cp2 = pltpu.make_async_copy(
                obs[0].at[pl.ds(0, 8)],
                dst.at[pl.ds(NSLAB * 16, 8)], wsems[0])
            cp2.start()
            cp2.wait()

    one_table(ut_t, u2)
    one_table(it_t, i2)


@jax.jit
def _sc_transpose(ut_t, it_t):
    mesh = plsc.VectorSubcoreMesh(core_axis_name="c", subcore_axis_name="s")
    f = pl.kernel(
        _tr_body,
        out_type=(
            jax.ShapeDtypeStruct((NROW // 8, 128), jnp.float32),
            jax.ShapeDtypeStruct((NROW // 8, 128), jnp.float32),
        ),
        mesh=mesh,
        compiler_params=pltpu.CompilerParams(use_tc_tiling_on_sc=True,
                                             needs_layout_passes=False),
        scratch_types=(
            [pltpu.VMEM((16, CHUNK), jnp.float32)] * 8
            + [pltpu.VMEM((16, TAIL), jnp.float32)]
            + [pltpu.SemaphoreType.DMA] * 9
        ),
    )
    return f(ut_t, it_t)


def _sc_body(uid_hbm, pos_hbm, neg_hbm, nbr_hbm, ut_hbm, it_hbm,
             user_out, pos_out, neg_out, nbr_out, plg_out, nlg_out, blg_out,
             idx_v, rows_a, rows_b, ptv, ut_v, lc0, lc1,
             sem_ga, sem_gb, sem_w, sem_l0, sem_l1):
    wid = lax.axis_index("s") * NC + lax.axis_index("c")
    i16 = _iota16()
    bufs = [rows_a, rows_b]
    gsems = [sem_ga, sem_gb]
    lcs = [lc0, lc1]
    lsems = [sem_l0, sem_l1]

    def g_start(table, j, b):
        pltpu.make_async_copy(table.at[idx_v.at[j]], bufs[b],
                              gsems[b]).start()

    def g_wait(table, b):
        pltpu.make_async_copy(table.at[idx_v.at[0]], bufs[b],
                              gsems[b]).wait()

    def drain_w(n8, src8):
        def d8(_, carry):
            pltpu.make_async_copy(src8.at[pl.ds(0, 8)],
                                  ptv.at[pl.ds(0, 8)], sem_w).wait()
            return carry
        lax.fori_loop(0, n8, d8, 0)

    pltpu.sync_copy(uid_hbm.at[wid], idx_v.at[pl.ds(0, 1)])
    g_start(ut_hbm, 0, 0)
    g_wait(ut_hbm, 0)

    def ugrp(g, carry):
        col = g * 16 + i16
        for d in range(D):
            v = plsc.load_gather(rows_a, [col, _splat(d)])
            plsc.store_scatter(ut_v, [_splat(d), col], v)
        return carry
    lax.fori_loop(0, 8, ugrp, 0)

    for ti in range(2):
        cp = pltpu.make_async_copy(
            ut_v.at[pl.ds(ti * 8, 8)],
            user_out.at[pl.ds(ti * 256 + wid * 8, 8)], sem_w)
        cp.start()
    drain_w(2, user_out)

    def seg(idx3, nch, ld, table, out_d, out_lg):
        pltpu.sync_copy(idx3.at[wid], idx_v.at[pl.ds(0, nch)])
        g_start(table, 0, 0)
        lg_base = wid * nch * CHUNK

        def process(j, rv, lc):
            def grp(g, c2):
                col = g * 16 + i16
                fr16 = j * CHUNK + col
                bl16 = fr16 // ld
                ll16 = (fr16 % ld) * 16
                acc = jnp.zeros((16,), jnp.float32)
                for d in range(D):
                    v = plsc.load_gather(rv, [col, _splat(d)])
                    u = plsc.load_gather(ut_v, [_splat(d), bl16])
                    plsc.store_scatter(ptv, [ll16 + d, bl16], v)
                    acc = acc + v * u
                lc[pl.ds(g * 16, 16)] = acc
                return c2
            lax.fori_loop(0, 8, grp, 0)

        def pair(t, carry):
            for b in range(2):
                j = 2 * t + b

                @pl.when(j + 1 < nch)
                def _():
                    g_start(table, j + 1, 1 - b)
                g_wait(table, b)

                @pl.when(t >= 1)
                def _():
                    pltpu.make_async_copy(lcs[b], out_lg.at[pl.ds(0, CHUNK)],
                                          lsems[b]).wait()
                process(j, bufs[b], lcs[b])
                pltpu.make_async_copy(
                    lcs[b], out_lg.at[pl.ds(lg_base + j * CHUNK, CHUNK)],
                    lsems[b]).start()
            return carry
        lax.fori_loop(0, nch // 2, pair, 0)
        for b in range(2):
            pltpu.make_async_copy(lcs[b], out_lg.at[pl.ds(0, CHUNK)],
                                  lsems[b]).wait()

        def wr_l(l, carry):
            for ti in range(2):
                cp = pltpu.make_async_copy(
                    ptv.at[pl.ds(l * 16 + ti * 8, 8)],
                    out_d.at[pl.ds((l * 2 + ti) * 256 + wid * 8, 8)], sem_w)
                cp.start()
            return carry
        lax.fori_loop(0, ld, wr_l, 0)
        drain_w(2 * ld, out_d)

    seg(nbr_hbm, NBR_CHUNKS, LN, ut_hbm, nbr_out, blg_out)
    seg(pos_hbm, POS_CHUNKS, L, it_hbm, pos_out, plg_out)
    seg(neg_hbm, POS_CHUNKS, L, it_hbm, neg_out, nlg_out)


@jax.jit
def _sc_fused(uid3, pos3, neg3, nbr3, user_table, item_table):
    mesh = plsc.VectorSubcoreMesh(core_axis_name="c", subcore_axis_name="s")
    f = pl.kernel(
        _sc_body,
        out_type=(
            jax.ShapeDtypeStruct((2 * 32 * 8, 128), jnp.float32),
            jax.ShapeDtypeStruct((L * 2 * 32 * 8, 128), jnp.float32),
            jax.ShapeDtypeStruct((L * 2 * 32 * 8, 128), jnp.float32),
            jax.ShapeDtypeStruct((LN * 2 * 32 * 8, 128), jnp.float32),
            jax.ShapeDtypeStruct((B * L,), jnp.float32),
            jax.ShapeDtypeStruct((B * L,), jnp.float32),
            jax.ShapeDtypeStruct((B * LN,), jnp.float32),
        ),
        mesh=mesh,
        compiler_params=pltpu.CompilerParams(use_tc_tiling_on_sc=False,
                                             needs_layout_passes=False),
        scratch_types=[
            pltpu.VMEM((POS_CHUNKS, CHUNK), jnp.int32),
            pltpu.VMEM((CHUNK, D), jnp.float32),
            pltpu.VMEM((CHUNK, D), jnp.float32),
            pltpu.VMEM((L * 16, 128), jnp.float32),
            pltpu.VMEM((16, 128), jnp.float32),
            pltpu.VMEM((CHUNK,), jnp.float32),
            pltpu.VMEM((CHUNK,), jnp.float32),
            pltpu.SemaphoreType.DMA,
            pltpu.SemaphoreType.DMA,
            pltpu.SemaphoreType.DMA,
            pltpu.SemaphoreType.DMA,
            pltpu.SemaphoreType.DMA,
        ],
    )
    return f(uid3, pos3, neg3, nbr3, user_table, item_table)


def kernel(uid, seq, pos, neg, nbr, nbr_iid, user_table, item_table):
    del seq, nbr_iid
    uid3 = uid.astype(jnp.int32).reshape(NW, 1, CHUNK)
    pos3 = pos.astype(jnp.int32).reshape(NW, POS_CHUNKS, CHUNK)
    neg3 = neg.astype(jnp.int32).reshape(NW, POS_CHUNKS, CHUNK)
    nbr3 = nbr.astype(jnp.int32).reshape(NW, NBR_CHUNKS, CHUNK)

    u2, i2 = _sc_transpose(user_table.T, item_table.T)
    ut_lin = u2.reshape(USER_NUM, D)
    it_lin = i2.reshape(ITEM_NUM, D)

    user_o, pos_o, neg_o, nbr_o, plg, nlg, blg = _sc_fused(
        uid3, pos3, neg3, nbr3, ut_lin, it_lin)

    user_emb = user_o.reshape(2, 32, 8, 128).transpose(1, 3, 0, 2) \
        .reshape(B, D)
    pos_hi = pos_o.reshape(L, 2, 32, 8, 128).transpose(2, 4, 0, 1, 3) \
        .reshape(B, L, D)
    neg_hi = neg_o.reshape(L, 2, 32, 8, 128).transpose(2, 4, 0, 1, 3) \
        .reshape(B, L, D)
    nbr_emb = nbr_o.reshape(LN, 2, 32, 8, 128).transpose(2, 4, 0, 1, 3) \
        .reshape(B, LN, D)
    pos_logits = plg.reshape(B, L)
    neg_logits = nlg.reshape(B, L)
    nbr_logits = blg.reshape(B, LN)

    return (pos_logits, neg_logits, nbr_logits, user_emb, pos_hi, neg_hi,
            nbr_emb)
